# baseline (speedup 1.0000x reference)
import functools

import jax
import jax.numpy as jnp
from jax import lax
from jax.experimental import pallas as pl
from jax.experimental.pallas import tpu as pltpu
from jax.experimental.pallas import tpu_sc as plsc

EMB = 64
NROW = 16384
NCOL = 200
BLK = 128
CHUNK = 256
CPJ = NROW // CHUNK
NCHUNKS = NCOL * CPJ
NB = 2

_info = plsc.get_sparse_core_info()
_NC, _NS = _info.num_cores, _info.num_subcores
_NW = _NC * _NS
CH_PER_W = NCHUNKS // _NW


def _make_gather():
  mesh = plsc.VectorSubcoreMesh(core_axis_name="c", subcore_axis_name="s")

  scratch = []
  for _ in range(NB):
    scratch.append(pltpu.VMEM((CHUNK,), jnp.int32))
    scratch.append(pltpu.VMEM((CHUNK, EMB), jnp.float32))
    scratch.append(pltpu.VMEM((1, 8, 1, 8, CHUNK), jnp.float32))
    scratch.append(pltpu.SemaphoreType.DMA)
    scratch.append(pltpu.SemaphoreType.DMA)

  @functools.partial(
      pl.kernel,
      mesh=mesh,
      compiler_params=pltpu.CompilerParams(
          use_tc_tiling_on_sc=False, needs_layout_passes=False
      ),
      out_type=jax.ShapeDtypeStruct((NCOL, 8, BLK, 8, BLK), jnp.float32),
      scratch_types=scratch,
  )
  def gather_kernel(zt_hbm, w_hbm, out_hbm, *bufs):
    idx_v = bufs[0::5]
    rows_v = bufs[1::5]
    trans_v = bufs[2::5]
    sem_g = bufs[3::5]
    sem_s = bufs[4::5]

    wid = lax.axis_index("s") * _NC + lax.axis_index("c")
    base = wid * CH_PER_W

    iota = jax.lax.iota(jnp.int32, 16)
    zero16 = jnp.zeros((16,), dtype=jnp.int32)
    lo3 = jnp.bitwise_and(iota, 7)
    hi8 = jnp.right_shift(iota, 3)

    def load_idx(p, u):
      j = u // CPJ
      m = u % CPJ
      pltpu.sync_copy(zt_hbm.at[j, pl.ds(m * CHUNK, CHUNK)], idx_v[p])

    def process_chunk(p, u):
      rows = rows_v[p]
      dst = trans_v[p]
      j = u // CPJ
      m = u % CPJ

      @plsc.parallel_loop(0, CHUNK, unroll=16)
      def _(ii):
        col = jnp.full((16,), 0, dtype=jnp.int32) + ii
        for h in range(EMB // 16):
          v = rows[ii, pl.ds(16 * h, 16)]
          plsc.store_scatter(dst, [zero16, hi8 + 2 * h, zero16, lo3, col], v)

      for bs in range(CHUNK // BLK):
        b = m * (CHUNK // BLK) + bs
        for t in range(8):
          pltpu.async_copy(
              trans_v[p].at[
                  pl.ds(0, 1), pl.ds(t, 1), pl.ds(0, 1), pl.ds(0, 8),
                  pl.ds(bs * BLK, BLK),
              ],
              out_hbm.at[pl.ds(j, 1), pl.ds(t, 1), pl.ds(b, 1)],
              sem_s[p],
          )

    def drain_stores(p, u):
      j = u // CPJ
      m = u % CPJ
      for bs in range(CHUNK // BLK):
        b = m * (CHUNK // BLK) + bs
        for t in range(8):
          pltpu.make_async_copy(
              trans_v[p].at[
                  pl.ds(0, 1), pl.ds(t, 1), pl.ds(0, 1), pl.ds(0, 8),
                  pl.ds(bs * BLK, BLK),
              ],
              out_hbm.at[pl.ds(j, 1), pl.ds(t, 1), pl.ds(b, 1)],
              sem_s[p],
          ).wait()

    for p in range(NB):
      load_idx(p, base + p)
      pltpu.async_copy(w_hbm.at[idx_v[p]], rows_v[p], sem_g[p])

    def ring_body(g, carry):
      for p in range(NB):
        n = g * NB + p
        u = base + n
        pltpu.make_async_copy(w_hbm.at[idx_v[p]], rows_v[p], sem_g[p]).wait()

        @pl.when(n + NB < CH_PER_W)
        def _():
          load_idx(p, u + NB)

        @pl.when(n >= NB)
        def _():
          drain_stores(p, u - NB)

        process_chunk(p, u)

        @pl.when(n + NB < CH_PER_W)
        def _():
          pltpu.async_copy(w_hbm.at[idx_v[p]], rows_v[p], sem_g[p])

      return carry

    lax.fori_loop(0, CH_PER_W // NB, ring_body, 0)

    for p in range(NB):
      drain_stores(p, base + CH_PER_W - NB + p)

  return gather_kernel


_gather = _make_gather()


def kernel(Z, W):
  zt = jnp.swapaxes(Z, 0, 1).astype(jnp.int32)
  out5 = _gather(zt, W)
  return out5.transpose(2, 4, 0, 1, 3).reshape(NROW, NCOL, EMB)


# speedup vs baseline: 1.0148x; 1.0148x over previous
import functools

import jax
import jax.numpy as jnp
from jax import lax
from jax.experimental import pallas as pl
from jax.experimental.pallas import tpu as pltpu
from jax.experimental.pallas import tpu_sc as plsc

EMB = 64
NROW = 16384
NCOL = 200
BLK = 128
CHUNK = 256
CPJ = NROW // CHUNK
NCHUNKS = NCOL * CPJ
NB = 2

_info = plsc.get_sparse_core_info()
_NC, _NS = _info.num_cores, _info.num_subcores
_NW = _NC * _NS
CH_PER_W = NCHUNKS // _NW


def _make_gather():
  mesh = plsc.VectorSubcoreMesh(core_axis_name="c", subcore_axis_name="s")

  scratch = []
  for _ in range(NB):
    scratch.append(pltpu.VMEM((CHUNK,), jnp.int32))
    scratch.append(pltpu.VMEM((CHUNK, EMB), jnp.float32))
    scratch.append(pltpu.VMEM((1, 8, 1, 8, CHUNK), jnp.float32))
    scratch.append(pltpu.SemaphoreType.DMA)
    scratch.append(pltpu.SemaphoreType.DMA)

  @functools.partial(
      pl.kernel,
      mesh=mesh,
      compiler_params=pltpu.CompilerParams(
          use_tc_tiling_on_sc=False, needs_layout_passes=False
      ),
      out_type=jax.ShapeDtypeStruct((NCOL, 8, BLK, 8, BLK), jnp.float32),
      scratch_types=scratch,
  )
  def gather_kernel(zt_hbm, w_hbm, out_hbm, *bufs):
    idx_v = bufs[0::5]
    rows_v = bufs[1::5]
    trans_v = bufs[2::5]
    sem_g = bufs[3::5]
    sem_s = bufs[4::5]

    wid = lax.axis_index("s") * _NC + lax.axis_index("c")
    base = wid * CH_PER_W

    iota = jax.lax.iota(jnp.int32, 16)
    zero16 = jnp.zeros((16,), dtype=jnp.int32)
    lo3 = jnp.bitwise_and(iota, 7)
    hi8 = jnp.right_shift(iota, 3)

    def load_idx(p, u):
      j = u // CPJ
      m = u % CPJ
      pltpu.sync_copy(zt_hbm.at[j, pl.ds(m * CHUNK, CHUNK)], idx_v[p])

    def process_chunk(p, u):
      rows = rows_v[p]
      dst = trans_v[p]
      j = u // CPJ
      m = u % CPJ

      flat_iota = iota * CHUNK

      @plsc.parallel_loop(0, CHUNK, unroll=8)
      def _(ii):
        for h in range(EMB // 16):
          v = rows[ii, pl.ds(16 * h, 16)]
          flat = flat_iota + (16 * h * CHUNK + ii)
          plsc.store_scatter(dst, [zero16, zero16, zero16, zero16, flat], v)

      for bs in range(CHUNK // BLK):
        b = m * (CHUNK // BLK) + bs
        for t in range(8):
          pltpu.async_copy(
              trans_v[p].at[
                  pl.ds(0, 1), pl.ds(t, 1), pl.ds(0, 1), pl.ds(0, 8),
                  pl.ds(bs * BLK, BLK),
              ],
              out_hbm.at[pl.ds(j, 1), pl.ds(t, 1), pl.ds(b, 1)],
              sem_s[p],
          )

    def drain_stores(p, u):
      j = u // CPJ
      m = u % CPJ
      for bs in range(CHUNK // BLK):
        b = m * (CHUNK // BLK) + bs
        for t in range(8):
          pltpu.make_async_copy(
              trans_v[p].at[
                  pl.ds(0, 1), pl.ds(t, 1), pl.ds(0, 1), pl.ds(0, 8),
                  pl.ds(bs * BLK, BLK),
              ],
              out_hbm.at[pl.ds(j, 1), pl.ds(t, 1), pl.ds(b, 1)],
              sem_s[p],
          ).wait()

    for p in range(NB):
      load_idx(p, base + p)
      pltpu.async_copy(w_hbm.at[idx_v[p]], rows_v[p], sem_g[p])

    def ring_body(g, carry):
      for p in range(NB):
        n = g * NB + p
        u = base + n
        pltpu.make_async_copy(w_hbm.at[idx_v[p]], rows_v[p], sem_g[p]).wait()

        @pl.when(n + NB < CH_PER_W)
        def _():
          load_idx(p, u + NB)

        @pl.when(n >= NB)
        def _():
          drain_stores(p, u - NB)

        process_chunk(p, u)

        @pl.when(n + NB < CH_PER_W)
        def _():
          pltpu.async_copy(w_hbm.at[idx_v[p]], rows_v[p], sem_g[p])

      return carry

    lax.fori_loop(0, CH_PER_W // NB, ring_body, 0)

    for p in range(NB):
      drain_stores(p, base + CH_PER_W - NB + p)

  return gather_kernel


_gather = _make_gather()


def kernel(Z, W):
  zt = jnp.swapaxes(Z, 0, 1).astype(jnp.int32)
  out5 = _gather(zt, W)
  return out5.transpose(2, 4, 0, 1, 3).reshape(NROW, NCOL, EMB)


# speedup vs baseline: 3.5022x; 3.4511x over previous
import functools

import jax
import jax.numpy as jnp
from jax import lax
from jax.experimental import pallas as pl
from jax.experimental.pallas import tpu as pltpu
from jax.experimental.pallas import tpu_sc as plsc

EMB = 64
NROW = 16384
NCOL = 200
BLK = 128
CHUNK = 256
CPJ = NROW // CHUNK
NCHUNKS = NCOL * CPJ
NB = 2

_info = plsc.get_sparse_core_info()
_NC, _NS = _info.num_cores, _info.num_subcores
_NW = _NC * _NS
CH_PER_W = NCHUNKS // _NW


def _make_gather():
  mesh = plsc.VectorSubcoreMesh(core_axis_name="c", subcore_axis_name="s")

  scratch = []
  for _ in range(NB):
    scratch.append(pltpu.VMEM((CHUNK,), jnp.int32))
    scratch.append(pltpu.VMEM((CHUNK, EMB + 1), jnp.float32))
    scratch.append(pltpu.VMEM((1, 8, 1, 8, CHUNK), jnp.float32))
    scratch.append(pltpu.SemaphoreType.DMA)
    scratch.append(pltpu.SemaphoreType.DMA)

  @functools.partial(
      pl.kernel,
      mesh=mesh,
      compiler_params=pltpu.CompilerParams(
          use_tc_tiling_on_sc=False, needs_layout_passes=False
      ),
      out_type=jax.ShapeDtypeStruct((NCOL, 8, BLK, 8, BLK), jnp.float32),
      scratch_types=scratch,
  )
  def gather_kernel(zt_hbm, w_hbm, out_hbm, *bufs):
    idx_v = bufs[0::5]
    rows_v = bufs[1::5]
    trans_v = bufs[2::5]
    sem_g = bufs[3::5]
    sem_s = bufs[4::5]

    wid = lax.axis_index("s") * _NC + lax.axis_index("c")
    base = wid * CH_PER_W

    iota = jax.lax.iota(jnp.int32, 16)
    zero16 = jnp.zeros((16,), dtype=jnp.int32)
    lo3 = jnp.bitwise_and(iota, 7)
    hi8 = jnp.right_shift(iota, 3)

    def load_idx(p, u):
      j = u // CPJ
      m = u % CPJ
      pltpu.sync_copy(zt_hbm.at[j, pl.ds(m * CHUNK, CHUNK)], idx_v[p])

    def process_chunk(p, u):
      rows = rows_v[p]
      dst = trans_v[p]
      j = u // CPJ
      m = u % CPJ

      @plsc.parallel_loop(0, EMB, unroll=4)
      def _(e):
        col = jnp.full((16,), 0, dtype=jnp.int32) + e
        t = jnp.right_shift(e, 3)
        r = jnp.bitwise_and(e, 7)
        for g in range(CHUNK // 16):
          v = plsc.load_gather(rows, [iota + 16 * g, col])
          dst[0, t, 0, r, pl.ds(16 * g, 16)] = v

      for bs in range(CHUNK // BLK):
        b = m * (CHUNK // BLK) + bs
        for t in range(8):
          pltpu.async_copy(
              trans_v[p].at[
                  pl.ds(0, 1), pl.ds(t, 1), pl.ds(0, 1), pl.ds(0, 8),
                  pl.ds(bs * BLK, BLK),
              ],
              out_hbm.at[pl.ds(j, 1), pl.ds(t, 1), pl.ds(b, 1)],
              sem_s[p],
          )

    def drain_stores(p, u):
      j = u // CPJ
      m = u % CPJ
      for bs in range(CHUNK // BLK):
        b = m * (CHUNK // BLK) + bs
        for t in range(8):
          pltpu.make_async_copy(
              trans_v[p].at[
                  pl.ds(0, 1), pl.ds(t, 1), pl.ds(0, 1), pl.ds(0, 8),
                  pl.ds(bs * BLK, BLK),
              ],
              out_hbm.at[pl.ds(j, 1), pl.ds(t, 1), pl.ds(b, 1)],
              sem_s[p],
          ).wait()

    for p in range(NB):
      load_idx(p, base + p)
      pltpu.async_copy(w_hbm.at[idx_v[p]], rows_v[p], sem_g[p])

    def ring_body(g, carry):
      for p in range(NB):
        n = g * NB + p
        u = base + n
        pltpu.make_async_copy(w_hbm.at[idx_v[p]], rows_v[p], sem_g[p]).wait()

        @pl.when(n + NB < CH_PER_W)
        def _():
          load_idx(p, u + NB)

        @pl.when(n >= NB)
        def _():
          drain_stores(p, u - NB)

        process_chunk(p, u)

        @pl.when(n + NB < CH_PER_W)
        def _():
          pltpu.async_copy(w_hbm.at[idx_v[p]], rows_v[p], sem_g[p])

      return carry

    lax.fori_loop(0, CH_PER_W // NB, ring_body, 0)

    for p in range(NB):
      drain_stores(p, base + CH_PER_W - NB + p)

  return gather_kernel


_gather = _make_gather()


def kernel(Z, W):
  zt = jnp.swapaxes(Z, 0, 1).astype(jnp.int32)
  wp = jnp.pad(W, ((0, 0), (0, 1)))
  out5 = _gather(zt, wp)
  return out5.transpose(2, 4, 0, 1, 3).reshape(NROW, NCOL, EMB)


# speedup vs baseline: 3.5023x; 1.0000x over previous
import functools

import jax
import jax.numpy as jnp
from jax import lax
from jax.experimental import pallas as pl
from jax.experimental.pallas import tpu as pltpu
from jax.experimental.pallas import tpu_sc as plsc

EMB = 64
NROW = 16384
NCOL = 200
BLK = 128
CHUNK = 256
CPJ = NROW // CHUNK
NCHUNKS = NCOL * CPJ
NB = 2

_info = plsc.get_sparse_core_info()
_NC, _NS = _info.num_cores, _info.num_subcores
_NW = _NC * _NS
CH_PER_W = NCHUNKS // _NW


def _make_gather():
  mesh = plsc.VectorSubcoreMesh(core_axis_name="c", subcore_axis_name="s")

  scratch = []
  for _ in range(NB):
    scratch.append(pltpu.VMEM((CHUNK,), jnp.int32))
    scratch.append(pltpu.VMEM((CHUNK, EMB + 8), jnp.float32))
    scratch.append(pltpu.VMEM((1, 8, 1, 8, CHUNK), jnp.float32))
    scratch.append(pltpu.SemaphoreType.DMA)
    scratch.append(pltpu.SemaphoreType.DMA)

  @functools.partial(
      pl.kernel,
      mesh=mesh,
      compiler_params=pltpu.CompilerParams(
          use_tc_tiling_on_sc=False, needs_layout_passes=False
      ),
      out_type=jax.ShapeDtypeStruct((NCOL, 8, BLK, 8, BLK), jnp.float32),
      scratch_types=scratch,
  )
  def gather_kernel(zt_hbm, w_hbm, out_hbm, *bufs):
    idx_v = bufs[0::5]
    rows_v = bufs[1::5]
    trans_v = bufs[2::5]
    sem_g = bufs[3::5]
    sem_s = bufs[4::5]

    wid = lax.axis_index("s") * _NC + lax.axis_index("c")
    base = wid * CH_PER_W

    iota = jax.lax.iota(jnp.int32, 16)
    zero16 = jnp.zeros((16,), dtype=jnp.int32)
    lo3 = jnp.bitwise_and(iota, 7)
    hi8 = jnp.right_shift(iota, 3)

    def load_idx(p, u):
      j = u // CPJ
      m = u % CPJ
      pltpu.sync_copy(zt_hbm.at[j, pl.ds(m * CHUNK, CHUNK)], idx_v[p])

    def process_chunk(p, u):
      rows = rows_v[p]
      dst = trans_v[p]
      j = u // CPJ
      m = u % CPJ

      @plsc.parallel_loop(0, EMB, unroll=4)
      def _(e):
        col = jnp.full((16,), 0, dtype=jnp.int32) + e
        t = jnp.right_shift(e, 3)
        r = jnp.bitwise_and(e, 7)
        for g in range(CHUNK // 16):
          v = plsc.load_gather(rows, [iota + 16 * g, col])
          dst[0, t, 0, r, pl.ds(16 * g, 16)] = v

      for bs in range(CHUNK // BLK):
        b = m * (CHUNK // BLK) + bs
        for t in range(8):
          pltpu.async_copy(
              trans_v[p].at[
                  pl.ds(0, 1), pl.ds(t, 1), pl.ds(0, 1), pl.ds(0, 8),
                  pl.ds(bs * BLK, BLK),
              ],
              out_hbm.at[pl.ds(j, 1), pl.ds(t, 1), pl.ds(b, 1)],
              sem_s[p],
          )

    def drain_stores(p, u):
      j = u // CPJ
      m = u % CPJ
      for bs in range(CHUNK // BLK):
        b = m * (CHUNK // BLK) + bs
        for t in range(8):
          pltpu.make_async_copy(
              trans_v[p].at[
                  pl.ds(0, 1), pl.ds(t, 1), pl.ds(0, 1), pl.ds(0, 8),
                  pl.ds(bs * BLK, BLK),
              ],
              out_hbm.at[pl.ds(j, 1), pl.ds(t, 1), pl.ds(b, 1)],
              sem_s[p],
          ).wait()

    for p in range(NB):
      load_idx(p, base + p)
      pltpu.async_copy(w_hbm.at[idx_v[p]], rows_v[p], sem_g[p])

    def ring_body(g, carry):
      for p in range(NB):
        n = g * NB + p
        u = base + n
        pltpu.make_async_copy(w_hbm.at[idx_v[p]], rows_v[p], sem_g[p]).wait()

        @pl.when(n + NB < CH_PER_W)
        def _():
          load_idx(p, u + NB)

        @pl.when(n >= NB)
        def _():
          drain_stores(p, u - NB)

        process_chunk(p, u)

        @pl.when(n + NB < CH_PER_W)
        def _():
          pltpu.async_copy(w_hbm.at[idx_v[p]], rows_v[p], sem_g[p])

      return carry

    lax.fori_loop(0, CH_PER_W // NB, ring_body, 0)

    for p in range(NB):
      drain_stores(p, base + CH_PER_W - NB + p)

  return gather_kernel


_gather = _make_gather()


def kernel(Z, W):
  zt = jnp.swapaxes(Z, 0, 1).astype(jnp.int32)
  wp = jnp.pad(W, ((0, 0), (0, 8)))
  out5 = _gather(zt, wp)
  return out5.transpose(2, 4, 0, 1, 3).reshape(NROW, NCOL, EMB)


# speedup vs baseline: 3.5082x; 1.0017x over previous
import functools

import jax
import jax.numpy as jnp
from jax import lax
from jax.experimental import pallas as pl
from jax.experimental.pallas import tpu as pltpu
from jax.experimental.pallas import tpu_sc as plsc

EMB = 64
NROW = 16384
NCOL = 200
BLK = 128
CHUNK = 256
CPJ = NROW // CHUNK
NCHUNKS = NCOL * CPJ
NB = 2

_info = plsc.get_sparse_core_info()
_NC, _NS = _info.num_cores, _info.num_subcores
_NW = _NC * _NS
CH_PER_W = NCHUNKS // _NW


def _make_gather():
  mesh = plsc.VectorSubcoreMesh(core_axis_name="c", subcore_axis_name="s")

  scratch = []
  for _ in range(NB):
    scratch.append(pltpu.VMEM((CHUNK,), jnp.int32))
    scratch.append(pltpu.VMEM((CHUNK, EMB + 8), jnp.float32))
    scratch.append(pltpu.VMEM((1, 8, 1, 8, CHUNK), jnp.float32))
    scratch.append(pltpu.SemaphoreType.DMA)
    scratch.append(pltpu.SemaphoreType.DMA)

  @functools.partial(
      pl.kernel,
      mesh=mesh,
      compiler_params=pltpu.CompilerParams(
          use_tc_tiling_on_sc=False, needs_layout_passes=False
      ),
      out_type=jax.ShapeDtypeStruct((NCOL, 8, BLK, 8, BLK), jnp.float32),
      scratch_types=scratch,
  )
  def gather_kernel(zt_hbm, w_hbm, out_hbm, *bufs):
    idx_v = bufs[0::5]
    rows_v = bufs[1::5]
    trans_v = bufs[2::5]
    sem_g = bufs[3::5]
    sem_s = bufs[4::5]

    wid = lax.axis_index("s") * _NC + lax.axis_index("c")
    base = wid * CH_PER_W

    iota = jax.lax.iota(jnp.int32, 16)

    def load_idx(p, u):
      j = u // CPJ
      m = u % CPJ
      pltpu.sync_copy(zt_hbm.at[j, pl.ds(m * CHUNK, CHUNK)], idx_v[p])

    def process_chunk(p, u):
      rows = rows_v[p]
      dst = trans_v[p]
      j = u // CPJ
      m = u % CPJ

      @plsc.parallel_loop(0, EMB, unroll=4)
      def _(e):
        col = jnp.full((16,), 0, dtype=jnp.int32) + e
        t = jnp.right_shift(e, 3)
        r = jnp.bitwise_and(e, 7)
        for g in range(CHUNK // 16):
          v = plsc.load_gather(rows, [iota + 16 * g, col])
          dst[0, t, 0, r, pl.ds(16 * g, 16)] = v

      for bs in range(CHUNK // BLK):
        b = m * (CHUNK // BLK) + bs
        for t in range(8):
          pltpu.async_copy(
              trans_v[p].at[
                  pl.ds(0, 1), pl.ds(t, 1), pl.ds(0, 1), pl.ds(0, 8),
                  pl.ds(bs * BLK, BLK),
              ],
              out_hbm.at[pl.ds(j, 1), pl.ds(t, 1), pl.ds(b, 1)],
              sem_s[p],
          )

    def drain_stores(p, u):
      j = u // CPJ
      m = u % CPJ
      for bs in range(CHUNK // BLK):
        b = m * (CHUNK // BLK) + bs
        for t in range(8):
          pltpu.make_async_copy(
              trans_v[p].at[
                  pl.ds(0, 1), pl.ds(t, 1), pl.ds(0, 1), pl.ds(0, 8),
                  pl.ds(bs * BLK, BLK),
              ],
              out_hbm.at[pl.ds(j, 1), pl.ds(t, 1), pl.ds(b, 1)],
              sem_s[p],
          ).wait()

    for p in range(NB):
      load_idx(p, base + p)
      pltpu.async_copy(w_hbm.at[idx_v[p]], rows_v[p], sem_g[p])

    def ring_body(g, carry):
      for p in range(NB):
        n = g * NB + p
        u = base + n
        pltpu.make_async_copy(w_hbm.at[idx_v[p]], rows_v[p], sem_g[p]).wait()

        @pl.when(n + NB < CH_PER_W)
        def _():
          load_idx(p, u + NB)

        @pl.when(n >= NB)
        def _():
          drain_stores(p, u - NB)

        process_chunk(p, u)

        @pl.when(n + NB < CH_PER_W)
        def _():
          pltpu.async_copy(w_hbm.at[idx_v[p]], rows_v[p], sem_g[p])

      return carry

    lax.fori_loop(0, CH_PER_W // NB, ring_body, 0)

    for p in range(NB):
      drain_stores(p, base + CH_PER_W - NB + p)

  return gather_kernel


_gather = _make_gather()


def kernel(Z, W):
  zt = jnp.swapaxes(Z, 0, 1).astype(jnp.int32)
  wp = jnp.pad(W, ((0, 0), (0, 8)))
  out5 = _gather(zt, wp)
  return out5.transpose(2, 4, 0, 1, 3).reshape(NROW, NCOL, EMB)
